# baseline (speedup 1.0000x reference)
import functools

import jax
import jax.numpy as jnp
from jax import lax
from jax.experimental import pallas as pl
from jax.experimental.pallas import tpu as pltpu
from jax.experimental.pallas import tpu_sc as plsc

_V = 1024
_D = 4096
_B = 8 * 1024

_GATHER_DNUMS = jax.lax.GatherDimensionNumbers(
    offset_dims=(), collapsed_slice_dims=(0,), start_index_map=(0,))


@functools.lru_cache(maxsize=None)
def _make_lookup(V, D, B):
    info = plsc.get_sparse_core_info()
    NC, NS, L = info.num_cores, info.num_subcores, info.num_lanes
    NW = NC * NS
    R = 16
    P = V // (NW * R)
    assert P * NW * R == V and B % L == 0
    n_groups = B // L
    mesh = plsc.VectorSubcoreMesh(core_axis_name="c", subcore_axis_name="s")

    @functools.partial(
        pl.kernel,
        mesh=mesh,
        out_type=jax.ShapeDtypeStruct((B, D), jnp.float32),
        scratch_types=[
            pltpu.VMEM((R, D), jnp.float32),
            pltpu.VMEM((B,), jnp.int32),
            pltpu.SemaphoreType.DMA,
        ],
    )
    def k(idx_hbm, table_hbm, out_hbm, tbl_v, idx_v, wsem):
        wid = lax.axis_index("s") * NC + lax.axis_index("c")
        pltpu.sync_copy(idx_hbm, idx_v)

        lane_iota = lax.iota(jnp.int32, L)
        step_masks = [((lane_iota - d) >> 31) + 1 for d in (1, 2, 4, 8)]

        def do_pass(p):
            lo = p * (NW * R) + wid * R
            pltpu.sync_copy(table_hbm.at[pl.ds(lo, R)], tbl_v)

            def scan_body(g, cnt):
                rel = idx_v[pl.ds(g * L, L)] - lo
                m01 = ((rel >> 31) + 1) * (-((rel - R) >> 31))
                csum = m01
                for d, mk in zip((1, 2, 4, 8), step_masks):
                    shifted = lax.gather(
                        csum,
                        jnp.maximum(lane_iota - d, 0)[:, None],
                        _GATHER_DNUMS, (1,),
                        mode=lax.GatherScatterMode.PROMISE_IN_BOUNDS)
                    csum = csum + shifted * mk
                c = csum[L - 1]

                @pl.when(c > 0)
                def _():
                    for lane in range(L):
                        r = rel[lane]

                        @pl.when((r >= 0) & (r < R))
                        def _():
                            pltpu.make_async_copy(
                                tbl_v.at[r], out_hbm.at[g * L + lane], wsem
                            ).start()

                return cnt + c

            n_match = lax.fori_loop(0, n_groups, scan_body, jnp.int32(0))

            def drain_body(j, carry):
                pltpu.make_async_copy(tbl_v.at[0], out_hbm.at[0], wsem).wait()
                return carry

            lax.fori_loop(0, n_match, drain_body, 0)

        for p in range(P):
            do_pass(p)

    return k


def kernel(indices, table):
    idx_flat = indices.reshape(-1).astype(jnp.int32)
    out = _make_lookup(_V, _D, _B)(idx_flat, table)
    return out.reshape(indices.shape[0], indices.shape[1], table.shape[1])


# speedup vs baseline: 1.3982x; 1.3982x over previous
import functools

import jax
import jax.numpy as jnp
from jax import lax
from jax.experimental import pallas as pl
from jax.experimental.pallas import tpu as pltpu
from jax.experimental.pallas import tpu_sc as plsc

_V = 1024
_D = 4096
_B = 8 * 1024


@functools.lru_cache(maxsize=None)
def _make_gather(V, D, B):
    info = plsc.get_sparse_core_info()
    NC, NS = info.num_cores, info.num_subcores
    NW = NC * NS
    assert B % (8 * NW) == 0
    b_per_w = B // NW
    C = 8
    n_chunks = b_per_w // C
    n_pairs = n_chunks // 2
    assert n_chunks == 2 * n_pairs and n_pairs >= 2
    mesh = plsc.VectorSubcoreMesh(core_axis_name="c", subcore_axis_name="s")

    @functools.partial(
        pl.kernel,
        mesh=mesh,
        out_type=jax.ShapeDtypeStruct((B, D), jnp.float32),
        scratch_types=[
            pltpu.VMEM((b_per_w,), jnp.int32),
            pltpu.VMEM((C, D), jnp.float32),
            pltpu.VMEM((C, D), jnp.float32),
            pltpu.SemaphoreType.DMA,
            pltpu.SemaphoreType.DMA,
            pltpu.SemaphoreType.DMA,
            pltpu.SemaphoreType.DMA,
        ],
    )
    def k(idx_hbm, table_hbm, out_hbm, idx_v, buf0, buf1, g0, g1, o0, o1):
        wid = lax.axis_index("s") * NC + lax.axis_index("c")
        base = wid * b_per_w
        pltpu.sync_copy(idx_hbm.at[pl.ds(base, b_per_w)], idx_v)

        def gather(i, buf, sem):
            return pltpu.make_async_copy(
                table_hbm.at[idx_v.at[pl.ds(i * C, C)]], buf, sem)

        def outcopy(i, buf, sem):
            return pltpu.make_async_copy(
                buf, out_hbm.at[pl.ds(base + i * C, C)], sem)

        gather(0, buf0, g0).start()

        def body(j, carry):
            i0 = 2 * j

            @pl.when(j >= 1)
            def _():
                outcopy(i0 - 1, buf1, o1).wait()

            gather(i0 + 1, buf1, g1).start()
            gather(i0, buf0, g0).wait()
            outcopy(i0, buf0, o0).start()

            @pl.when(j + 1 < n_pairs)
            def _():
                outcopy(i0, buf0, o0).wait()
                gather(i0 + 2, buf0, g0).start()

            gather(i0 + 1, buf1, g1).wait()
            outcopy(i0 + 1, buf1, o1).start()
            return carry

        lax.fori_loop(0, n_pairs, body, 0)
        outcopy(n_chunks - 2, buf0, o0).wait()
        outcopy(n_chunks - 1, buf1, o1).wait()

    return k


def kernel(indices, table):
    idx_flat = indices.reshape(-1).astype(jnp.int32)
    out = _make_gather(_V, _D, _B)(idx_flat, table)
    return out.reshape(indices.shape[0], indices.shape[1], table.shape[1])
